# Initial kernel scaffold; baseline (speedup 1.0000x reference)
import jax
import jax.numpy as jnp
from jax.experimental import pallas as pl


def kernel(x, indices):
    raise NotImplementedError("write your pallas kernel here")


# speedup vs baseline: 1.0415x; 1.0415x over previous
import functools

import jax
import jax.numpy as jnp
from jax import lax
from jax.experimental import pallas as pl
from jax.experimental.pallas import tpu as pltpu
from jax.experimental.pallas import tpu_sc as plsc

_D = 64
_M = 2145
_B = 8192
_NC = 2
_NS = 16
_NW = _NC * _NS
_ROWS_PER_W = _B // _NW
_SUB = 8
_NCHUNK = _ROWS_PER_W // _SUB
_XBLK = _SUB * _D
_ONES = _XBLK
_OUTW = _SUB * _M
_NFULL = _M // 16
_LAST = _M - 16
_MPAD = 2160
_BIG = 1 << 20


def _make_poly():
    mesh = plsc.VectorSubcoreMesh(core_axis_name="c", subcore_axis_name="s")

    @functools.partial(
        pl.kernel,
        mesh=mesh,
        out_type=jax.ShapeDtypeStruct((_B * _M,), jnp.float32),
        compiler_params=pltpu.CompilerParams(needs_layout_passes=False),
        scratch_types=[
            pltpu.VMEM((_XBLK + 16,), jnp.float32),
            pltpu.VMEM((_MPAD,), jnp.int32),
            pltpu.VMEM((_MPAD,), jnp.int32),
            pltpu.VMEM((_OUTW,), jnp.float32),
        ],
    )
    def _poly(x_hbm, ti_hbm, tj_hbm, out_hbm, x_v, ti_v, tj_v, out_v):
        wid = lax.axis_index("s") * _NC + lax.axis_index("c")
        pltpu.sync_copy(ti_hbm, ti_v)
        pltpu.sync_copy(tj_hbm, tj_v)
        x_v[pl.ds(_ONES, 16)] = jnp.full((16,), 1.0, jnp.float32)
        row_base = wid * _ROWS_PER_W

        def chunk_body(chunk, _):
            row0 = row_base + chunk * _SUB
            pltpu.sync_copy(x_hbm.at[pl.ds(row0 * _D, _XBLK)],
                            x_v.at[pl.ds(0, _XBLK)])

            def do_chunk(co):
                vi = ti_v[pl.ds(co, 16)]
                vj = tj_v[pl.ds(co, 16)]
                for r in range(_SUB):
                    fi = jnp.minimum(vi + r * _D, _ONES)
                    fj = jnp.minimum(vj + r * _D, _ONES)
                    a = plsc.load_gather(x_v, [fi])
                    b = plsc.load_gather(x_v, [fj])
                    out_v[pl.ds(r * _M + co, 16)] = a * b

            def c_body(c, _):
                do_chunk(c * 16)
                return 0

            lax.fori_loop(0, _NFULL, c_body, 0)
            do_chunk(_LAST)
            pltpu.sync_copy(out_v, out_hbm.at[pl.ds(row0 * _M, _OUTW)])
            return 0

        lax.fori_loop(0, _NCHUNK, chunk_body, 0)

    return _poly


_poly_call = _make_poly()


def kernel(x, indices):
    x_flat = x.reshape(-1).astype(jnp.float32)
    ii = indices[:, 0].astype(jnp.int32)
    jj = indices[:, 1].astype(jnp.int32)
    ti = jnp.where(ii >= _D, _BIG, ii)
    tj = jnp.where(jj >= _D, _BIG, jj)
    pad = jnp.full((_MPAD - _M,), _BIG, jnp.int32)
    ti = jnp.concatenate([ti, pad])
    tj = jnp.concatenate([tj, pad])
    out = _poly_call(x_flat, ti, tj)
    return out.reshape(_B, _M)


# speedup vs baseline: 1.8368x; 1.7636x over previous
import functools

import jax
import jax.numpy as jnp
from jax import lax
from jax.experimental import pallas as pl
from jax.experimental.pallas import tpu as pltpu
from jax.experimental.pallas import tpu_sc as plsc

_D = 64
_M = 2145
_B = 8192
_NC = 2
_NS = 16
_NW = _NC * _NS
_ROWS_PER_W = _B // _NW
_SUB = 8
_NCHUNK = _ROWS_PER_W // _SUB
_XBLK = _SUB * _D
_ONES = _XBLK
_OUTW = _SUB * _M
_NCH16 = _M // 16 + 1
_LAST = _M - 16
_MPAD = 2160
_BIG = 1 << 20


def _make_poly():
    mesh = plsc.VectorSubcoreMesh(core_axis_name="c", subcore_axis_name="s")

    @functools.partial(
        pl.kernel,
        mesh=mesh,
        out_type=jax.ShapeDtypeStruct((_B * _M,), jnp.float32),
        compiler_params=pltpu.CompilerParams(needs_layout_passes=False),
        scratch_types=[
            pltpu.VMEM((_XBLK + 16,), jnp.float32),
            pltpu.VMEM((_XBLK + 16,), jnp.float32),
            pltpu.VMEM((_MPAD,), jnp.int32),
            pltpu.VMEM((_MPAD,), jnp.int32),
            pltpu.VMEM((_OUTW,), jnp.float32),
            pltpu.VMEM((_OUTW,), jnp.float32),
            pltpu.SemaphoreType.DMA,
            pltpu.SemaphoreType.DMA,
            pltpu.SemaphoreType.DMA,
            pltpu.SemaphoreType.DMA,
        ],
    )
    def _poly(x_hbm, ti_hbm, tj_hbm, out_hbm, x0, x1, ti_v, tj_v,
              o0, o1, sx0, sx1, so0, so1):
        wid = lax.axis_index("s") * _NC + lax.axis_index("c")
        pltpu.sync_copy(ti_hbm, ti_v)
        pltpu.sync_copy(tj_hbm, tj_v)
        x0[pl.ds(_ONES, 16)] = jnp.full((16,), 1.0, jnp.float32)
        x1[pl.ds(_ONES, 16)] = jnp.full((16,), 1.0, jnp.float32)
        row_base = wid * _ROWS_PER_W
        xs = (x0, x1)
        os_ = (o0, o1)
        sxs = (sx0, sx1)
        sos = (so0, so1)

        def start_x(chunk, buf, sem):
            ch = jnp.minimum(chunk, _NCHUNK - 1)
            row0 = row_base + ch * _SUB
            pltpu.async_copy(x_hbm.at[pl.ds(row0 * _D, _XBLK)],
                             buf.at[pl.ds(0, _XBLK)], sem)

        def wait_x(buf, sem):
            pltpu.make_async_copy(x_hbm.at[pl.ds(0, _XBLK)],
                                  buf.at[pl.ds(0, _XBLK)], sem).wait()

        def compute(x_v, out_v):
            @plsc.parallel_loop(0, _NCH16, unroll=4)
            def _(c):
                co = jnp.minimum(c * 16, _LAST)
                vi = ti_v[pl.ds(co, 16)]
                vj = tj_v[pl.ds(co, 16)]
                for r in range(_SUB):
                    fi = jnp.minimum(vi + r * _D, _ONES)
                    fj = jnp.minimum(vj + r * _D, _ONES)
                    a = plsc.load_gather(x_v, [fi])
                    b = plsc.load_gather(x_v, [fj])
                    out_v[pl.ds(r * _M + co, 16)] = a * b

        def start_out(chunk, buf, sem):
            row0 = row_base + chunk * _SUB
            pltpu.async_copy(buf, out_hbm.at[pl.ds(row0 * _M, _OUTW)], sem)

        def wait_out(buf, sem):
            pltpu.make_async_copy(buf, out_hbm.at[pl.ds(0, _OUTW)], sem).wait()

        start_x(0, x0, sx0)
        start_x(1, x1, sx1)
        for p in range(2):
            wait_x(xs[p], sxs[p])
            compute(xs[p], os_[p])
            start_out(jnp.int32(p), os_[p], sos[p])
            start_x(jnp.int32(p + 2), xs[p], sxs[p])

        def pair_body(k, _):
            for p in range(2):
                ch = 2 * k + p
                wait_x(xs[p], sxs[p])
                wait_out(os_[p], sos[p])
                compute(xs[p], os_[p])
                start_out(ch, os_[p], sos[p])
                start_x(ch + 2, xs[p], sxs[p])
            return 0

        lax.fori_loop(1, _NCHUNK // 2, pair_body, 0)

        for p in range(2):
            wait_x(xs[p], sxs[p])
            wait_out(os_[p], sos[p])

    return _poly


_poly_call = _make_poly()


def kernel(x, indices):
    x_flat = x.reshape(-1).astype(jnp.float32)
    ii = indices[:, 0].astype(jnp.int32)
    jj = indices[:, 1].astype(jnp.int32)
    ti = jnp.where(ii >= _D, _BIG, ii)
    tj = jnp.where(jj >= _D, _BIG, jj)
    pad = jnp.full((_MPAD - _M,), _BIG, jnp.int32)
    ti = jnp.concatenate([ti, pad])
    tj = jnp.concatenate([tj, pad])
    out = _poly_call(x_flat, ti, tj)
    return out.reshape(_B, _M)


# speedup vs baseline: 3.0817x; 1.6777x over previous
import functools

import jax
import jax.numpy as jnp
from jax import lax
from jax.experimental import pallas as pl
from jax.experimental.pallas import tpu as pltpu
from jax.experimental.pallas import tpu_sc as plsc

_D = 64
_M = 2145
_B = 8192
_NC = 2
_NS = 16
_NW = _NC * _NS
_ROWS_PER_W = _B // _NW
_SUB = 8
_NCHUNK = _ROWS_PER_W // _SUB
_XBLK = _SUB * _D
_ONES = _XBLK
_OUTW = _SUB * _M
_NCH16 = _M // 16 + 1
_LAST = _M - 16
_MPAD = 2160
_BIG = 1 << 20


def _make_poly():
    mesh = plsc.VectorSubcoreMesh(core_axis_name="c", subcore_axis_name="s")

    @functools.partial(
        pl.kernel,
        mesh=mesh,
        out_type=jax.ShapeDtypeStruct((_B, _M), jnp.float32),
        compiler_params=pltpu.CompilerParams(needs_layout_passes=False),
        scratch_types=[
            pltpu.VMEM((_XBLK + 16,), jnp.float32),
            pltpu.VMEM((_XBLK + 16,), jnp.float32),
            pltpu.VMEM((_MPAD,), jnp.int32),
            pltpu.VMEM((_MPAD,), jnp.int32),
            pltpu.VMEM((_SUB, _M), jnp.float32),
            pltpu.VMEM((_SUB, _M), jnp.float32),
            pltpu.SemaphoreType.DMA,
            pltpu.SemaphoreType.DMA,
            pltpu.SemaphoreType.DMA,
            pltpu.SemaphoreType.DMA,
        ],
    )
    def _poly(x_hbm, ti_hbm, tj_hbm, out_hbm, x0, x1, ti_v, tj_v,
              o0, o1, sx0, sx1, so0, so1):
        wid = lax.axis_index("s") * _NC + lax.axis_index("c")
        pltpu.sync_copy(ti_hbm, ti_v)
        pltpu.sync_copy(tj_hbm, tj_v)
        x0[pl.ds(_ONES, 16)] = jnp.full((16,), 1.0, jnp.float32)
        x1[pl.ds(_ONES, 16)] = jnp.full((16,), 1.0, jnp.float32)
        row_base = wid * _ROWS_PER_W
        xs = (x0, x1)
        os_ = (o0, o1)
        sxs = (sx0, sx1)
        sos = (so0, so1)

        def start_x(chunk, buf, sem):
            ch = jnp.minimum(chunk, _NCHUNK - 1)
            row0 = row_base + ch * _SUB
            pltpu.async_copy(x_hbm.at[pl.ds(row0 * _D, _XBLK)],
                             buf.at[pl.ds(0, _XBLK)], sem)

        def wait_x(buf, sem):
            pltpu.make_async_copy(x_hbm.at[pl.ds(0, _XBLK)],
                                  buf.at[pl.ds(0, _XBLK)], sem).wait()

        def compute(x_v, out_v):
            @plsc.parallel_loop(0, _NCH16, unroll=4)
            def _(c):
                co = jnp.minimum(c * 16, _LAST)
                vi = ti_v[pl.ds(co, 16)]
                vj = tj_v[pl.ds(co, 16)]
                for r in range(_SUB):
                    fi = jnp.minimum(vi + r * _D, _ONES)
                    fj = jnp.minimum(vj + r * _D, _ONES)
                    a = plsc.load_gather(x_v, [fi])
                    b = plsc.load_gather(x_v, [fj])
                    out_v[r, pl.ds(co, 16)] = a * b

        def start_out(chunk, buf, sem):
            row0 = row_base + chunk * _SUB
            pltpu.async_copy(buf, out_hbm.at[pl.ds(row0, _SUB), :], sem)

        def wait_out(buf, sem):
            pltpu.make_async_copy(buf, out_hbm.at[pl.ds(0, _SUB), :], sem).wait()

        start_x(0, x0, sx0)
        start_x(1, x1, sx1)
        for p in range(2):
            wait_x(xs[p], sxs[p])
            compute(xs[p], os_[p])
            start_out(jnp.int32(p), os_[p], sos[p])
            start_x(jnp.int32(p + 2), xs[p], sxs[p])

        def pair_body(k, _):
            for p in range(2):
                ch = 2 * k + p
                wait_x(xs[p], sxs[p])
                wait_out(os_[p], sos[p])
                compute(xs[p], os_[p])
                start_out(ch, os_[p], sos[p])
                start_x(ch + 2, xs[p], sxs[p])
            return 0

        lax.fori_loop(1, _NCHUNK // 2, pair_body, 0)

        for p in range(2):
            wait_x(xs[p], sxs[p])
            wait_out(os_[p], sos[p])

    return _poly


_poly_call = _make_poly()


def kernel(x, indices):
    x_flat = x.reshape(-1).astype(jnp.float32)
    ii = indices[:, 0].astype(jnp.int32)
    jj = indices[:, 1].astype(jnp.int32)
    ti = jnp.where(ii >= _D, _BIG, ii)
    tj = jnp.where(jj >= _D, _BIG, jj)
    pad = jnp.full((_MPAD - _M,), _BIG, jnp.int32)
    ti = jnp.concatenate([ti, pad])
    tj = jnp.concatenate([tj, pad])
    return _poly_call(x_flat, ti, tj)


# speedup vs baseline: 3.2140x; 1.0429x over previous
import functools

import jax
import jax.numpy as jnp
from jax import lax
from jax.experimental import pallas as pl
from jax.experimental.pallas import tpu as pltpu
from jax.experimental.pallas import tpu_sc as plsc

_D = 64
_M = 2145
_MP = 2152
_B = 8192
_NC = 2
_NS = 16
_NW = _NC * _NS
_BW = _B // _NW
_NK = _BW // 16
_NMT = _MP // 8


def _make_poly():
    mesh = plsc.VectorSubcoreMesh(core_axis_name="c", subcore_axis_name="s")

    @functools.partial(
        pl.kernel,
        mesh=mesh,
        out_type=jax.ShapeDtypeStruct((_MP, _B), jnp.float32),
        compiler_params=pltpu.CompilerParams(needs_layout_passes=False),
        scratch_types=[
            pltpu.VMEM((_D + 1, _BW), jnp.float32),
            pltpu.VMEM((_MP,), jnp.int32),
            pltpu.VMEM((_MP,), jnp.int32),
            pltpu.VMEM((8, _BW), jnp.float32),
            pltpu.VMEM((8, _BW), jnp.float32),
            pltpu.SemaphoreType.DMA,
            pltpu.SemaphoreType.DMA,
        ],
    )
    def _poly(xt_hbm, ti_hbm, tj_hbm, out_hbm,
              x_v, ti_v, tj_v, o0, o1, s0, s1):
        wid = lax.axis_index("s") * _NC + lax.axis_index("c")
        b0 = wid * _BW
        pltpu.sync_copy(ti_hbm, ti_v)
        pltpu.sync_copy(tj_hbm, tj_v)
        pltpu.sync_copy(xt_hbm.at[:, pl.ds(b0, _BW)], x_v.at[pl.ds(0, _D), :])
        ones16 = jnp.full((16,), 1.0, jnp.float32)
        for k in range(_NK):
            x_v[_D, pl.ds(k * 16, 16)] = ones16
        os_ = (o0, o1)
        sems = (s0, s1)
        lane = lax.iota(jnp.int32, 16)

        def compute(mt, buf):
            ivs, jvs = [], []
            for m_r in range(8):
                mvec = jnp.full((16,), jnp.int32(0), jnp.int32) + (mt * 8 + m_r)
                ivs.append(plsc.load_gather(ti_v, [mvec]))
                jvs.append(plsc.load_gather(tj_v, [mvec]))

            @plsc.parallel_loop(0, _NK, unroll=2)
            def _(k):
                colv = lane + k * 16
                for m_r in range(8):
                    a = plsc.load_gather(x_v, [ivs[m_r], colv])
                    b = plsc.load_gather(x_v, [jvs[m_r], colv])
                    buf[m_r, pl.ds(k * 16, 16)] = a * b

        def start_out(mt, buf, sem):
            pltpu.async_copy(buf, out_hbm.at[pl.ds(mt * 8, 8), pl.ds(b0, _BW)],
                             sem)

        def wait_out(buf, sem):
            pltpu.make_async_copy(buf, out_hbm.at[pl.ds(0, 8), pl.ds(0, _BW)],
                                  sem).wait()

        for p in range(2):
            compute(jnp.int32(p), os_[p])
            if p == 0:
                for k in range(_NK):
                    os_[p][0, pl.ds(k * 16, 16)] = ones16
            start_out(jnp.int32(p), os_[p], sems[p])

        def pair_body(k, _):
            for p in range(2):
                mt = 2 * k + p
                wait_out(os_[p], sems[p])
                compute(mt, os_[p])
                start_out(mt, os_[p], sems[p])
            return 0

        lax.fori_loop(1, _NMT // 2, pair_body, 0)

        wait_out(os_[0], sems[0])
        compute(jnp.int32(_NMT - 1), os_[0])
        start_out(jnp.int32(_NMT - 1), os_[0], sems[0])
        wait_out(os_[1], sems[1])
        wait_out(os_[0], sems[0])

    return _poly


_poly_call = _make_poly()


def kernel(x, indices):
    xt = x.T.astype(jnp.float32)
    pad = _MP - _M
    ti = jnp.pad(indices[:, 0].astype(jnp.int32), (0, pad))
    tj = jnp.pad(indices[:, 1].astype(jnp.int32), (0, pad))
    out = _poly_call(xt, ti, tj)
    return out[:_M].T
